# baseline (speedup 1.0000x reference)
import functools

import jax
import jax.numpy as jnp
from jax import lax
from jax.experimental import pallas as pl
from jax.experimental.pallas import tpu as pltpu
from jax.experimental.pallas import tpu_sc as plsc

VOCAB = 100000
EMB = 128
HID = 768
EPS = 1e-12

_NC = 2
_NS = 16
_NW = _NC * _NS


def _make_sc_gather(batch: int, seq: int, emb: int):
    n_tokens = batch * seq
    per_w = n_tokens // _NW
    w_per_row = seq // per_w
    nch = 4
    ch = per_w // nch
    mesh = plsc.VectorSubcoreMesh(core_axis_name="c", subcore_axis_name="s")

    @functools.partial(
        pl.kernel,
        mesh=mesh,
        out_type=jax.ShapeDtypeStruct((n_tokens, emb), jnp.float32),
        scratch_types=[
            pltpu.VMEM((per_w,), jnp.int32),
            pltpu.VMEM((per_w, emb), jnp.float32),
            pltpu.SemaphoreType.DMA,
            pltpu.SemaphoreType.DMA,
        ],
    )
    def gather_kernel(ids_hbm, table_hbm, out_hbm, idx_v, rows_v, gsem, ssem):
        wid = lax.axis_index("s") * _NC + lax.axis_index("c")
        row = wid // w_per_row
        col = (wid % w_per_row) * per_w
        base = wid * per_w
        pltpu.sync_copy(ids_hbm.at[row, pl.ds(col, per_w)], idx_v)
        gets = [
            pltpu.async_copy(
                table_hbm.at[idx_v.at[pl.ds(c * ch, ch)]],
                rows_v.at[pl.ds(c * ch, ch)], gsem)
            for c in range(nch)
        ]
        puts = []
        for c in range(nch):
            gets[c].wait()
            puts.append(pltpu.async_copy(
                rows_v.at[pl.ds(c * ch, ch)],
                out_hbm.at[pl.ds(base + c * ch, ch)], ssem))
        for p in puts:
            p.wait()

    return gather_kernel


def _tc_body(g_ref, tt_ref, te_ref, w_ref, b_ref, pos_ref, s_ref, o_ref):
    g = g_ref[...]
    tt = tt_ref[...].astype(jnp.float32)
    t0 = te_ref[0:1, :]
    t1 = te_ref[1:2, :]
    x = g + t0 + tt * (t1 - t0)
    y = jnp.dot(x.astype(jnp.bfloat16), w_ref[...].astype(jnp.bfloat16),
                preferred_element_type=jnp.float32)
    y = y + b_ref[...] + pos_ref[...]
    var = jnp.mean(y * y, axis=-1, keepdims=True)
    o_ref[...] = y * lax.rsqrt(var + EPS) * s_ref[...]


def kernel(input_ids, token_type_ids, word_emb, type_emb, W, b, pos_emb,
           scale):
    B, S = input_ids.shape
    N = B * S

    gathered = _make_sc_gather(B, S, EMB)(input_ids.astype(jnp.int32),
                                          word_emb)

    TB = 512
    seq_blocks = S // TB
    tti = token_type_ids.reshape(N, 1).astype(jnp.int32)
    tok_map = lambda j, i: (i * seq_blocks + j, 0)
    const_map = lambda j, i: (0, 0)
    out = pl.pallas_call(
        _tc_body,
        grid=(seq_blocks, B),
        in_specs=[
            pl.BlockSpec((TB, EMB), tok_map),
            pl.BlockSpec((TB, 1), tok_map),
            pl.BlockSpec((2, EMB), const_map),
            pl.BlockSpec((EMB, HID), const_map),
            pl.BlockSpec((1, HID), const_map),
            pl.BlockSpec((TB, HID), lambda j, i: (j, 0)),
            pl.BlockSpec((1, HID), const_map),
        ],
        out_specs=pl.BlockSpec((TB, HID), tok_map),
        out_shape=jax.ShapeDtypeStruct((N, HID), jnp.float32),
    )(gathered, tti, type_emb, W, b.reshape(1, HID), pos_emb,
      scale.reshape(1, HID))

    return out.reshape(B, S, HID)


# speedup vs baseline: 1.1100x; 1.1100x over previous
import functools

import jax
import jax.numpy as jnp
from jax import lax
from jax.experimental import pallas as pl
from jax.experimental.pallas import tpu as pltpu
from jax.experimental.pallas import tpu_sc as plsc

VOCAB = 100000
EMB = 128
HID = 768
EPS = 1e-12

_NC = 2
_NS = 16
_NW = _NC * _NS


def _make_sc_gather(batch: int, seq: int, emb: int):
    n_tokens = batch * seq
    per_w = n_tokens // _NW
    w_per_row = seq // per_w
    nch = 4
    ch = per_w // nch
    mesh = plsc.VectorSubcoreMesh(core_axis_name="c", subcore_axis_name="s")

    @functools.partial(
        pl.kernel,
        mesh=mesh,
        out_type=jax.ShapeDtypeStruct((n_tokens, emb), jnp.float32),
        scratch_types=[
            pltpu.VMEM((per_w,), jnp.int32),
            pltpu.VMEM((per_w, emb), jnp.float32),
            pltpu.SemaphoreType.DMA,
            pltpu.SemaphoreType.DMA,
        ],
    )
    def gather_kernel(ids_hbm, table_hbm, out_hbm, idx_v, rows_v, gsem, ssem):
        wid = lax.axis_index("s") * _NC + lax.axis_index("c")
        row = wid // w_per_row
        col = (wid % w_per_row) * per_w
        base = wid * per_w
        pltpu.sync_copy(ids_hbm.at[row, pl.ds(col, per_w)], idx_v)
        gets = [
            pltpu.async_copy(
                table_hbm.at[idx_v.at[pl.ds(c * ch, ch)]],
                rows_v.at[pl.ds(c * ch, ch)], gsem)
            for c in range(nch)
        ]
        puts = []
        for c in range(nch):
            gets[c].wait()
            puts.append(pltpu.async_copy(
                rows_v.at[pl.ds(c * ch, ch)],
                out_hbm.at[pl.ds(base + c * ch, ch)], ssem))
        for p in puts:
            p.wait()

    return gather_kernel


def _tc_body(seq_blocks, g_ref, tt_ref, te_ref, w_ref, b_ref, pos_ref, s_ref,
             o_ref):
    i = pl.program_id(0)
    g = g_ref[...]
    tt = tt_ref[...].astype(jnp.float32)
    t0 = te_ref[0:1, :]
    t1 = te_ref[1:2, :]
    x = g + t0 + tt * (t1 - t0)
    y = jnp.dot(x.astype(jnp.bfloat16), w_ref[...].astype(jnp.bfloat16),
                preferred_element_type=jnp.float32)
    tb = g.shape[0]
    pos_start = (i % seq_blocks) * tb
    y = y + b_ref[...] + pos_ref[pl.ds(pos_start, tb), :]
    var = jnp.mean(y * y, axis=-1, keepdims=True)
    o_ref[...] = y * lax.rsqrt(var + EPS) * s_ref[...]


def kernel(input_ids, token_type_ids, word_emb, type_emb, W, b, pos_emb,
           scale):
    B, S = input_ids.shape
    N = B * S

    gathered = _make_sc_gather(B, S, EMB)(input_ids.astype(jnp.int32),
                                          word_emb)

    TB = 1024
    seq_blocks = S // TB
    tti = token_type_ids.reshape(N, 1).astype(jnp.int32)
    out = pl.pallas_call(
        functools.partial(_tc_body, seq_blocks),
        grid=(N // TB,),
        in_specs=[
            pl.BlockSpec((TB, EMB), lambda i: (i, 0)),
            pl.BlockSpec((TB, 1), lambda i: (i, 0)),
            pl.BlockSpec((2, EMB), lambda i: (0, 0)),
            pl.BlockSpec((EMB, HID), lambda i: (0, 0)),
            pl.BlockSpec((1, HID), lambda i: (0, 0)),
            pl.BlockSpec((S, HID), lambda i: (0, 0)),
            pl.BlockSpec((1, HID), lambda i: (0, 0)),
        ],
        out_specs=pl.BlockSpec((TB, HID), lambda i: (i, 0)),
        out_shape=jax.ShapeDtypeStruct((N, HID), jnp.float32),
    )(gathered, tti, type_emb, W, b.reshape(1, HID), pos_emb,
      scale.reshape(1, HID))

    return out.reshape(B, S, HID)


# speedup vs baseline: 1.1604x; 1.0455x over previous
import functools

import jax
import jax.numpy as jnp
from jax import lax
from jax.experimental import pallas as pl
from jax.experimental.pallas import tpu as pltpu
from jax.experimental.pallas import tpu_sc as plsc

VOCAB = 100000
EMB = 128
HID = 768
EPS = 1e-12

_NC = 2
_NS = 16
_NW = _NC * _NS


def _make_sc_gather(batch: int, seq: int, emb: int):
    n_tokens = batch * seq
    per_w = n_tokens // _NW
    w_per_row = seq // per_w
    nch = 4
    ch = per_w // nch
    mesh = plsc.VectorSubcoreMesh(core_axis_name="c", subcore_axis_name="s")

    @functools.partial(
        pl.kernel,
        mesh=mesh,
        out_type=jax.ShapeDtypeStruct((n_tokens, emb), jnp.float32),
        scratch_types=[
            pltpu.VMEM((per_w,), jnp.int32),
            pltpu.VMEM((per_w, emb), jnp.float32),
            pltpu.SemaphoreType.DMA,
            pltpu.SemaphoreType.DMA,
        ],
    )
    def gather_kernel(ids_hbm, table_hbm, out_hbm, idx_v, rows_v, gsem, ssem):
        wid = lax.axis_index("s") * _NC + lax.axis_index("c")
        row = wid // w_per_row
        col = (wid % w_per_row) * per_w
        base = wid * per_w
        pltpu.sync_copy(ids_hbm.at[row, pl.ds(col, per_w)], idx_v)
        gets = [
            pltpu.async_copy(
                table_hbm.at[idx_v.at[pl.ds(c * ch, ch)]],
                rows_v.at[pl.ds(c * ch, ch)], gsem)
            for c in range(nch)
        ]
        puts = []
        for c in range(nch):
            gets[c].wait()
            puts.append(pltpu.async_copy(
                rows_v.at[pl.ds(c * ch, ch)],
                out_hbm.at[pl.ds(base + c * ch, ch)], ssem))
        for p in puts:
            p.wait()

    return gather_kernel


def _tc_body(seq_blocks, g_ref, tt_ref, te_ref, w_ref, b_ref, pos_ref, s_ref,
             o_ref):
    i = pl.program_id(0)
    g = g_ref[...]
    tt = tt_ref[...].astype(jnp.float32)
    t0 = te_ref[0:1, :]
    t1 = te_ref[1:2, :]
    x = g + t0 + tt * (t1 - t0)
    y = jnp.dot(x.astype(jnp.bfloat16), w_ref[...].astype(jnp.bfloat16),
                preferred_element_type=jnp.float32)
    tb = g.shape[0]
    pos_start = (i % seq_blocks) * tb
    y = y + b_ref[...] + pos_ref[pl.ds(pos_start, tb), :]
    var = jnp.mean(y * y, axis=-1, keepdims=True)
    o_ref[...] = y * lax.rsqrt(var + EPS) * s_ref[...]


def kernel(input_ids, token_type_ids, word_emb, type_emb, W, b, pos_emb,
           scale):
    B, S = input_ids.shape
    N = B * S

    gathered = _make_sc_gather(B, S, EMB)(input_ids.astype(jnp.int32),
                                          word_emb)

    TB = 2048
    seq_blocks = S // TB
    tti = token_type_ids.reshape(N, 1).astype(jnp.int32)
    out = pl.pallas_call(
        functools.partial(_tc_body, seq_blocks),
        grid=(N // TB,),
        in_specs=[
            pl.BlockSpec((TB, EMB), lambda i: (i, 0)),
            pl.BlockSpec((TB, 1), lambda i: (i, 0)),
            pl.BlockSpec((2, EMB), lambda i: (0, 0)),
            pl.BlockSpec((EMB, HID), lambda i: (0, 0)),
            pl.BlockSpec((1, HID), lambda i: (0, 0)),
            pl.BlockSpec((S, HID), lambda i: (0, 0)),
            pl.BlockSpec((1, HID), lambda i: (0, 0)),
        ],
        out_specs=pl.BlockSpec((TB, HID), lambda i: (i, 0)),
        out_shape=jax.ShapeDtypeStruct((N, HID), jnp.float32),
    )(gathered, tti, type_emb, W, b.reshape(1, HID), pos_emb,
      scale.reshape(1, HID))

    return out.reshape(B, S, HID)
